# Initial kernel scaffold; baseline (speedup 1.0000x reference)
import jax
import jax.numpy as jnp
from jax.experimental import pallas as pl


def kernel(x, edge_index):
    raise NotImplementedError("write your pallas kernel here")


# speedup vs baseline: 4.1439x; 4.1439x over previous
import functools

import jax
import jax.numpy as jnp
from jax import lax
from jax.experimental import pallas as pl
from jax.experimental.pallas import tpu as pltpu
from jax.experimental.pallas import tpu_sc as plsc

N = 10000
E = 320000
D = 128

NC = 2
NS = 16
NW = NC * NS
CHUNK = 128
N_PAD = 10240
RPT = N_PAD // NS
ZB = RPT // CHUNK

_MESH = dict(core_axis_name="c", subcore_axis_name="s")


def _sc_sum_kernel(x, src, dst):
    e_pad = src.shape[0]
    chunks_per_tile = e_pad // (NW * CHUNK)

    @functools.partial(
        pl.kernel,
        mesh=plsc.VectorSubcoreMesh(**_MESH),
        out_type=jax.ShapeDtypeStruct((NC, N_PAD, D), jnp.float32),
        scratch_types=[
            pltpu.VMEM((CHUNK,), jnp.int32),
            pltpu.VMEM((CHUNK,), jnp.int32),
            pltpu.VMEM((CHUNK, D), jnp.float32),
            pltpu.VMEM_SHARED((N_PAD, D), jnp.float32),
            pltpu.SemaphoreType.DMA,
        ],
    )
    def k(x_hbm, src_hbm, dst_hbm, psum_hbm,
          idx_s, idx_d, rows_v, acc_sh, sem):
        c = lax.axis_index("c")
        s = lax.axis_index("s")
        wid = c * NS + s
        z16 = jnp.zeros((16,), jnp.float32)

        def fill_rows(r, carry):
            for jj in range(D // 16):
                rows_v[r, pl.ds(jj * 16, 16)] = z16
            return carry

        lax.fori_loop(0, CHUNK, fill_rows, 0)
        for z in range(ZB):
            pltpu.sync_copy(rows_v, acc_sh.at[pl.ds((s * ZB + z) * CHUNK, CHUNK)])
        plsc.subcore_barrier()

        def body(i, carry):
            base = (wid * chunks_per_tile + i) * CHUNK
            pltpu.sync_copy(src_hbm.at[pl.ds(base, CHUNK)], idx_s)
            pltpu.async_copy(x_hbm.at[idx_s], rows_v, sem).wait()
            pltpu.sync_copy(dst_hbm.at[pl.ds(base, CHUNK)], idx_d)
            pltpu.sync_copy(rows_v, acc_sh.at[idx_d], add=True)
            return carry

        lax.fori_loop(0, chunks_per_tile, body, 0)
        plsc.subcore_barrier()

        for z in range(ZB):
            r0 = (s * ZB + z) * CHUNK
            pltpu.sync_copy(acc_sh.at[pl.ds(r0, CHUNK)], rows_v)
            pltpu.sync_copy(rows_v, psum_hbm.at[c, pl.ds(r0, CHUNK)])

    return k(x, src, dst)


def _sc_count_kernel(dst):
    e_pad = dst.shape[0]
    chunks_per_tile = e_pad // (NW * CHUNK)

    @functools.partial(
        pl.kernel,
        mesh=plsc.VectorSubcoreMesh(**_MESH),
        out_type=jax.ShapeDtypeStruct((NC, N_PAD, D), jnp.float32),
        scratch_types=[
            pltpu.VMEM((CHUNK,), jnp.int32),
            pltpu.VMEM((CHUNK, D), jnp.float32),
            pltpu.VMEM((CHUNK, D), jnp.float32),
            pltpu.VMEM_SHARED((N_PAD, D), jnp.float32),
        ],
    )
    def k(dst_hbm, pcnt_hbm, idx_d, ones_v, buf_v, cnt_sh):
        c = lax.axis_index("c")
        s = lax.axis_index("s")
        wid = c * NS + s
        z16 = jnp.zeros((16,), jnp.float32)
        o16 = jnp.ones((16,), jnp.float32)

        def fill(r, carry):
            for jj in range(D // 16):
                ones_v[r, pl.ds(jj * 16, 16)] = o16
                buf_v[r, pl.ds(jj * 16, 16)] = z16
            return carry

        lax.fori_loop(0, CHUNK, fill, 0)
        for z in range(ZB):
            pltpu.sync_copy(buf_v, cnt_sh.at[pl.ds((s * ZB + z) * CHUNK, CHUNK)])
        plsc.subcore_barrier()

        def body(i, carry):
            base = (wid * chunks_per_tile + i) * CHUNK
            pltpu.sync_copy(dst_hbm.at[pl.ds(base, CHUNK)], idx_d)
            pltpu.sync_copy(ones_v, cnt_sh.at[idx_d], add=True)
            return carry

        lax.fori_loop(0, chunks_per_tile, body, 0)
        plsc.subcore_barrier()

        for z in range(ZB):
            r0 = (s * ZB + z) * CHUNK
            pltpu.sync_copy(cnt_sh.at[pl.ds(r0, CHUNK)], buf_v)
            pltpu.sync_copy(buf_v, pcnt_hbm.at[c, pl.ds(r0, CHUNK)])

    return k(dst)


def _combine_kernel(psum, pcnt):
    BN = 2048

    def comb(ps_ref, pc_ref, o_ref):
        ssum = ps_ref[0] + ps_ref[1]
        cnt = pc_ref[0, :, 0:1] + pc_ref[1, :, 0:1]
        o_ref[...] = ssum / jnp.maximum(cnt, 1.0)

    return pl.pallas_call(
        comb,
        grid=(N_PAD // BN,),
        in_specs=[
            pl.BlockSpec((NC, BN, D), lambda i: (0, i, 0)),
            pl.BlockSpec((NC, BN, D), lambda i: (0, i, 0)),
        ],
        out_specs=pl.BlockSpec((BN, D), lambda i: (i, 0)),
        out_shape=jax.ShapeDtypeStruct((N_PAD, D), jnp.float32),
    )(psum, pcnt)


@jax.jit
def kernel(x, edge_index):
    src = edge_index[0]
    dst = edge_index[1]
    e_pad = ((E + NW * CHUNK - 1) // (NW * CHUNK)) * (NW * CHUNK)
    pad = e_pad - E
    src_p = jnp.concatenate([src, jnp.zeros((pad,), jnp.int32)])
    dst_p = jnp.concatenate([dst, jnp.full((pad,), N, jnp.int32)])
    psum = _sc_sum_kernel(x, src_p, dst_p)
    pcnt = _sc_count_kernel(dst_p)
    return _combine_kernel(psum, pcnt)[:N]
